# baseline (speedup 1.0000x reference)
import functools

import jax
import jax.numpy as jnp
from jax import lax
from jax.experimental import pallas as pl
from jax.experimental.pallas import tpu as pltpu
from jax.experimental.pallas import tpu_sc as plsc

_N = 10000
_E = 320000
_D_IN = 128
_D_H = 256
_D_OUT = 128

_NC = 2
_NS = 16
_NW = _NC * _NS
_CHUNK = 64
_SDB = 32
_NACC = 10240
_RPT = _NACC // _NS
_CNT_BASE = 10112
_PAD_ROW = _NACC - 1

_CH1 = 320
_EPW1 = _CH1 * _CHUNK
_EPAD1 = _EPW1 * _NW
_CH2 = 160
_EPW2 = _CH2 * _CHUNK
_EPAD2 = _EPW2 * _NW


def _seg_sum_body(chunks, feat, sd_i, zrows, out_feat,
                  sdb0, sdb1, rows0, rows1, acc,
                  sb0, sb1, sg0, sg1, ss0, ss1):
    c = lax.axis_index("c")
    s = lax.axis_index("s")
    wid = s * _NC + c
    cbase = wid * chunks
    nb = chunks // _SDB
    sdb = (sdb0, sdb1)
    rows = (rows0, rows1)
    sb = (sb0, sb1)
    sg = (sg0, sg1)
    ss = (ss0, ss1)

    def block_copy(q, k):
        pltpu.async_copy(sd_i.at[pl.ds((cbase + k * _SDB), _SDB)], sdb[q], sb[q])

    def block_wait(q, k):
        pltpu.make_async_copy(sd_i.at[pl.ds((cbase + k * _SDB), _SDB)],
                              sdb[q], sb[q]).wait()

    def gather_issue(b, q, t):
        pltpu.async_copy(feat.at[sdb[q].at[t, 0]], rows[b], sg[b])

    def gather_wait(b, q, t):
        pltpu.make_async_copy(feat.at[sdb[q].at[t, 0]], rows[b], sg[b]).wait()

    def scatter_issue(b, q, t):
        pltpu.async_copy(rows[b], acc.at[sdb[q].at[t, 1]], ss[b], add=True)

    def scatter_wait(b, q, t):
        pltpu.make_async_copy(rows[b], acc.at[sdb[q].at[t, 1]], ss[b]).wait()

    pltpu.sync_copy(zrows, acc.at[pl.ds(s * _RPT, _RPT)])
    plsc.subcore_barrier()

    block_copy(0, 0)
    block_wait(0, 0)
    gather_issue(0, 0, 0)
    for k in range(nb):
        q = k % 2
        qn = 1 - q
        if k + 1 < nb:
            block_copy(qn, k + 1)

        def pair(p, carry):
            t0 = p * 2
            gather_wait(0, q, t0)
            scatter_issue(0, q, t0)

            @pl.when(p > 0)
            def _():
                scatter_wait(1, q, t0 - 1)

            gather_issue(1, q, t0 + 1)
            gather_wait(1, q, t0 + 1)
            scatter_issue(1, q, t0 + 1)

            @pl.when(p < _SDB // 2 - 1)
            def _():
                scatter_wait(0, q, t0)
                gather_issue(0, q, t0 + 2)

            return carry

        lax.fori_loop(0, _SDB // 2, pair, 0)
        scatter_wait(0, q, _SDB - 2)
        scatter_wait(1, q, _SDB - 1)
        if k + 1 < nb:
            block_wait(qn, k + 1)
            gather_issue(0, qn, 0)

    plsc.subcore_barrier()
    pltpu.sync_copy(acc.at[pl.ds(s * _RPT, _RPT)],
                    out_feat.at[c].at[pl.ds(s * _RPT, _RPT)])


def _make_seg_sum(chunks):
    mesh = plsc.VectorSubcoreMesh(core_axis_name="c", subcore_axis_name="s")
    return pl.kernel(
        functools.partial(_seg_sum_body, chunks),
        out_type=[jax.ShapeDtypeStruct((_NC, _NACC, 128), jnp.float32)],
        mesh=mesh,
        scratch_types=[
            pltpu.VMEM((_SDB, 2, _CHUNK), jnp.int32),
            pltpu.VMEM((_SDB, 2, _CHUNK), jnp.int32),
            pltpu.VMEM((_CHUNK, 128), jnp.float32),
            pltpu.VMEM((_CHUNK, 128), jnp.float32),
            pltpu.VMEM_SHARED((_NACC, 128), jnp.float32),
            pltpu.SemaphoreType.DMA,
            pltpu.SemaphoreType.DMA,
            pltpu.SemaphoreType.DMA,
            pltpu.SemaphoreType.DMA,
            pltpu.SemaphoreType.DMA,
            pltpu.SemaphoreType.DMA,
        ],
    )


_seg_sum_l1 = _make_seg_sum(_CH1)
_seg_sum_l2 = _make_seg_sum(_CH2)

_BM = 1000


def _layer1_body(h0, s1, c1, wl1, b1, wr1, a1, wl2, wr2, b2, p2, r2):
    ssum = s1[0] + s1[1]
    cnt = jnp.maximum(c1[0] + c1[1], 1.0)
    agg = ssum / cnt
    dn = (((1,), (1,)), ((), ()))
    z = lax.dot_general(agg, wl1[...], dn, preferred_element_type=jnp.float32)
    z = z + lax.dot_general(h0[...], wr1[...], dn,
                            preferred_element_type=jnp.float32)
    z = z + b1[...]
    av = a1[0, 0]
    h1 = jnp.where(z >= 0.0, z, av * z)
    p2[...] = lax.dot_general(h1, wl2[...], dn,
                              preferred_element_type=jnp.float32)
    r2[...] = lax.dot_general(h1, wr2[...], dn,
                              preferred_element_type=jnp.float32) + b2[...]


_layer1 = pl.pallas_call(
    _layer1_body,
    grid=(_N // _BM,),
    in_specs=[
        pl.BlockSpec((_BM, _D_IN), lambda i: (i, 0)),
        pl.BlockSpec((_NC, _BM, 128), lambda i: (0, i, 0)),
        pl.BlockSpec((_NC, _BM, 1), lambda i: (0, i, 0)),
        pl.BlockSpec((_D_H, _D_IN), lambda i: (0, 0)),
        pl.BlockSpec((1, _D_H), lambda i: (0, 0)),
        pl.BlockSpec((_D_H, _D_IN), lambda i: (0, 0)),
        pl.BlockSpec((1, 1), lambda i: (0, 0)),
        pl.BlockSpec((_D_OUT, _D_H), lambda i: (0, 0)),
        pl.BlockSpec((_D_OUT, _D_H), lambda i: (0, 0)),
        pl.BlockSpec((1, _D_OUT), lambda i: (0, 0)),
    ],
    out_specs=[
        pl.BlockSpec((_BM, _D_OUT), lambda i: (i, 0)),
        pl.BlockSpec((_BM, _D_OUT), lambda i: (i, 0)),
    ],
    out_shape=[
        jax.ShapeDtypeStruct((_N, _D_OUT), jnp.float32),
        jax.ShapeDtypeStruct((_N, _D_OUT), jnp.float32),
    ],
)


def _layer2_body(s2, c1, r2, a2, out):
    ssum = s2[0] + s2[1]
    cnt = jnp.maximum(c1[0] + c1[1], 1.0)
    z = ssum / cnt + r2[...]
    av = a2[0, 0]
    out[...] = jnp.where(z >= 0.0, z, av * z)


_layer2 = pl.pallas_call(
    _layer2_body,
    grid=(_N // _BM,),
    in_specs=[
        pl.BlockSpec((_NC, _BM, _D_OUT), lambda i: (0, i, 0)),
        pl.BlockSpec((_NC, _BM, 1), lambda i: (0, i, 0)),
        pl.BlockSpec((_BM, _D_OUT), lambda i: (i, 0)),
        pl.BlockSpec((1, 1), lambda i: (0, 0)),
    ],
    out_specs=pl.BlockSpec((_BM, _D_OUT), lambda i: (i, 0)),
    out_shape=jax.ShapeDtypeStruct((_N, _D_OUT), jnp.float32),
)


def kernel(x, edge_index, edge_weights, edge_attr,
           W_l1, b_l1, W_r1, a1, W_l2, b_l2, W_r2, a2):
    h0 = x[0]
    ei = edge_index[0]
    src = ei[:, 0]
    dst = ei[:, 1]

    tab = jnp.concatenate([h0, jnp.eye(128, dtype=jnp.float32)], axis=0)
    src_cnt = _N + jnp.bitwise_and(dst, 127)
    dst_cnt = _CNT_BASE + jnp.right_shift(dst, 7)

    pad1 = _EPAD1 - 2 * _E
    src1 = jnp.concatenate([src, src_cnt, jnp.zeros((pad1,), jnp.int32)])
    dst1 = jnp.concatenate([dst, dst_cnt, jnp.full((pad1,), _PAD_ROW, jnp.int32)])
    sd1 = jnp.stack([src1.reshape(-1, _CHUNK), dst1.reshape(-1, _CHUNK)], axis=1)
    pad2 = _EPAD2 - _E
    src2 = jnp.concatenate([src, jnp.zeros((pad2,), jnp.int32)])
    dst2 = jnp.concatenate([dst, jnp.full((pad2,), _PAD_ROW, jnp.int32)])
    sd2 = jnp.stack([src2.reshape(-1, _CHUNK), dst2.reshape(-1, _CHUNK)], axis=1)

    zrows = jnp.zeros((_RPT, 128), jnp.float32)

    (s1,) = _seg_sum_l1(tab, sd1, zrows)
    cnt = s1[:, _CNT_BASE:_CNT_BASE + 80, :].reshape(_NC, 10240)
    cnt = cnt[:, :_N].reshape(_NC, _N, 1)
    p2, r2 = _layer1(h0, s1, cnt, W_l1, b_l1.reshape(1, -1), W_r1,
                     a1.reshape(1, 1), W_l2, W_r2, b_l2.reshape(1, -1))
    (s2,) = _seg_sum_l2(p2, sd2, zrows)
    h2 = _layer2(s2, cnt, r2, a2.reshape(1, 1))
    return h2.reshape(1, -1)


# speedup vs baseline: 1.7708x; 1.7708x over previous
import functools

import jax
import jax.numpy as jnp
from jax import lax
from jax.experimental import pallas as pl
from jax.experimental.pallas import tpu as pltpu
from jax.experimental.pallas import tpu_sc as plsc

_N = 10000
_E = 320000
_D_IN = 128
_D_H = 256
_D_OUT = 128

_NC = 2
_NS = 16
_NW = _NC * _NS
_CHUNK = 128
_NACC = 10240
_RPT = _NACC // _NS
_CNT_BASE = 10112
_PAD_ROW = _NACC - 1

_CH1 = 157
_EPW1 = _CH1 * _CHUNK
_EPAD1 = _EPW1 * _NW
_CH2 = 79
_EPW2 = _CH2 * _CHUNK
_EPAD2 = _EPW2 * _NW


def _seg_sum_body(chunks, feat, sd_i, zrows, out_feat, sdv, rows, acc, sem):
    c = lax.axis_index("c")
    s = lax.axis_index("s")
    wid = s * _NC + c
    cbase = wid * chunks

    pltpu.sync_copy(zrows, acc.at[pl.ds(s * _RPT, _RPT)])
    plsc.subcore_barrier()

    def step(j, carry):
        pltpu.sync_copy(sd_i.at[cbase + j], sdv)
        pltpu.async_copy(feat.at[sdv.at[0]], rows, sem).wait()
        pltpu.sync_copy(rows, acc.at[sdv.at[1]], add=True)
        return carry

    lax.fori_loop(0, chunks, step, 0)
    plsc.subcore_barrier()

    pltpu.sync_copy(acc.at[pl.ds(s * _RPT, _RPT)],
                    out_feat.at[c].at[pl.ds(s * _RPT, _RPT)])


def _make_seg_sum(chunks):
    mesh = plsc.VectorSubcoreMesh(core_axis_name="c", subcore_axis_name="s")
    return pl.kernel(
        functools.partial(_seg_sum_body, chunks),
        out_type=[jax.ShapeDtypeStruct((_NC, _NACC, 128), jnp.float32)],
        mesh=mesh,
        scratch_types=[
            pltpu.VMEM((2, _CHUNK), jnp.int32),
            pltpu.VMEM((_CHUNK, 128), jnp.float32),
            pltpu.VMEM_SHARED((_NACC, 128), jnp.float32),
            pltpu.SemaphoreType.DMA,
        ],
    )


_seg_sum_l1 = _make_seg_sum(_CH1)
_seg_sum_l2 = _make_seg_sum(_CH2)

_BM = 1000


def _layer1_body(h0, s1, c1, wl1, b1, wr1, a1, wl2, wr2, b2, p2, r2):
    ssum = s1[0] + s1[1]
    cnt = jnp.maximum(c1[0] + c1[1], 1.0)
    agg = ssum / cnt
    dn = (((1,), (1,)), ((), ()))
    z = lax.dot_general(agg, wl1[...], dn, preferred_element_type=jnp.float32)
    z = z + lax.dot_general(h0[...], wr1[...], dn,
                            preferred_element_type=jnp.float32)
    z = z + b1[...]
    av = a1[0, 0]
    h1 = jnp.where(z >= 0.0, z, av * z)
    p2[...] = lax.dot_general(h1, wl2[...], dn,
                              preferred_element_type=jnp.float32)
    r2[...] = lax.dot_general(h1, wr2[...], dn,
                              preferred_element_type=jnp.float32) + b2[...]


_layer1 = pl.pallas_call(
    _layer1_body,
    grid=(_N // _BM,),
    in_specs=[
        pl.BlockSpec((_BM, _D_IN), lambda i: (i, 0)),
        pl.BlockSpec((_NC, _BM, 128), lambda i: (0, i, 0)),
        pl.BlockSpec((_NC, _BM, 1), lambda i: (0, i, 0)),
        pl.BlockSpec((_D_H, _D_IN), lambda i: (0, 0)),
        pl.BlockSpec((1, _D_H), lambda i: (0, 0)),
        pl.BlockSpec((_D_H, _D_IN), lambda i: (0, 0)),
        pl.BlockSpec((1, 1), lambda i: (0, 0)),
        pl.BlockSpec((_D_OUT, _D_H), lambda i: (0, 0)),
        pl.BlockSpec((_D_OUT, _D_H), lambda i: (0, 0)),
        pl.BlockSpec((1, _D_OUT), lambda i: (0, 0)),
    ],
    out_specs=[
        pl.BlockSpec((_BM, _D_OUT), lambda i: (i, 0)),
        pl.BlockSpec((_BM, _D_OUT), lambda i: (i, 0)),
    ],
    out_shape=[
        jax.ShapeDtypeStruct((_N, _D_OUT), jnp.float32),
        jax.ShapeDtypeStruct((_N, _D_OUT), jnp.float32),
    ],
)


def _layer2_body(s2, c1, r2, a2, out):
    ssum = s2[0] + s2[1]
    cnt = jnp.maximum(c1[0] + c1[1], 1.0)
    z = ssum / cnt + r2[...]
    av = a2[0, 0]
    out[...] = jnp.where(z >= 0.0, z, av * z)


_layer2 = pl.pallas_call(
    _layer2_body,
    grid=(_N // _BM,),
    in_specs=[
        pl.BlockSpec((_NC, _BM, _D_OUT), lambda i: (0, i, 0)),
        pl.BlockSpec((_NC, _BM, 1), lambda i: (0, i, 0)),
        pl.BlockSpec((_BM, _D_OUT), lambda i: (i, 0)),
        pl.BlockSpec((1, 1), lambda i: (0, 0)),
    ],
    out_specs=pl.BlockSpec((_BM, _D_OUT), lambda i: (i, 0)),
    out_shape=jax.ShapeDtypeStruct((_N, _D_OUT), jnp.float32),
)


def kernel(x, edge_index, edge_weights, edge_attr,
           W_l1, b_l1, W_r1, a1, W_l2, b_l2, W_r2, a2):
    h0 = x[0]
    ei = edge_index[0]
    src = ei[:, 0]
    dst = ei[:, 1]

    tab = jnp.concatenate([h0, jnp.eye(128, dtype=jnp.float32)], axis=0)
    src_cnt = _N + jnp.bitwise_and(dst, 127)
    dst_cnt = _CNT_BASE + jnp.right_shift(dst, 7)

    pad1 = _EPAD1 - 2 * _E
    src1 = jnp.concatenate([src, src_cnt, jnp.zeros((pad1,), jnp.int32)])
    dst1 = jnp.concatenate([dst, dst_cnt, jnp.full((pad1,), _PAD_ROW, jnp.int32)])
    sd1 = jnp.stack([src1.reshape(-1, _CHUNK), dst1.reshape(-1, _CHUNK)], axis=1)
    pad2 = _EPAD2 - _E
    src2 = jnp.concatenate([src, jnp.zeros((pad2,), jnp.int32)])
    dst2 = jnp.concatenate([dst, jnp.full((pad2,), _PAD_ROW, jnp.int32)])
    sd2 = jnp.stack([src2.reshape(-1, _CHUNK), dst2.reshape(-1, _CHUNK)], axis=1)

    zrows = jnp.zeros((_RPT, 128), jnp.float32)

    (s1,) = _seg_sum_l1(tab, sd1, zrows)
    cnt = s1[:, _CNT_BASE:_CNT_BASE + 80, :].reshape(_NC, 10240)
    cnt = cnt[:, :_N].reshape(_NC, _N, 1)
    p2, r2 = _layer1(h0, s1, cnt, W_l1, b_l1.reshape(1, -1), W_r1,
                     a1.reshape(1, 1), W_l2, W_r2, b_l2.reshape(1, -1))
    (s2,) = _seg_sum_l2(p2, sd2, zrows)
    h2 = _layer2(s2, cnt, r2, a2.reshape(1, 1))
    return h2.reshape(1, -1)
